# baseline (speedup 1.0000x reference)
import functools

import jax
import jax.numpy as jnp
from jax import lax
from jax.experimental import pallas as pl
from jax.experimental.pallas import tpu as pltpu
from jax.experimental.pallas import tpu_sc as plsc

NUM_USERS = 1000000
NUM_ITEMS = 100000
EMBED_DIM = 32
BATCH = 16384

NC = 2
NS = 16
NW = NC * NS
BPW = BATCH // NW
CHUNK = 128
NCHUNK = BPW // CHUNK
GPC = CHUNK // 16

_mesh = plsc.VectorSubcoreMesh(core_axis_name="c", subcore_axis_name="s")


@functools.partial(
    pl.kernel,
    out_type=jax.ShapeDtypeStruct((BATCH,), jnp.float32),
    mesh=_mesh,
    compiler_params=pltpu.CompilerParams(needs_layout_passes=False,
                                         use_tc_tiling_on_sc=False),
    scratch_types=[
        pltpu.VMEM((NCHUNK, CHUNK), jnp.int32),
        pltpu.VMEM((NCHUNK, CHUNK), jnp.int32),
        pltpu.VMEM((NCHUNK, CHUNK), jnp.int32),
        pltpu.VMEM((NCHUNK, CHUNK), jnp.int32),
        pltpu.VMEM((BPW,), jnp.int32),
        pltpu.VMEM((BPW,), jnp.int32),
        pltpu.VMEM((CHUNK, 128), jnp.float32),
        pltpu.VMEM((CHUNK, 128), jnp.float32),
        pltpu.VMEM((BPW,), jnp.float32),
        pltpu.VMEM((BPW,), jnp.float32),
        pltpu.VMEM((16,), jnp.float32),
        pltpu.VMEM((BPW,), jnp.float32),
        pltpu.SemaphoreType.DMA,
        pltpu.SemaphoreType.DMA,
    ],
)
def _mf_sc(uids_hbm, iids_hbm, utab_hbm, itab_hbm, ub_hbm, ib_hbm, gb_hbm,
           out_hbm, uid_v, iid_v, ublk_v, iblk_v, ul_v, il_v, urows_v,
           irows_v, ubv_v, ibv_v, gb_v, out_v, sem, bsem):
    wid = lax.axis_index("s") * NC + lax.axis_index("c")
    base = pl.multiple_of(wid * BPW, BPW)

    pltpu.sync_copy(uids_hbm.at[wid], uid_v)
    pltpu.sync_copy(iids_hbm.at[wid], iid_v)
    pltpu.sync_copy(gb_hbm, gb_v)

    iota16 = lax.iota(jnp.int32, 16)

    for k in range(BPW // 16):
        s = k * 16
        ch, off = s // CHUNK, s % CHUNK
        u = uid_v[ch, pl.ds(off, 16)]
        i = iid_v[ch, pl.ds(off, 16)]
        ublk_v[ch, pl.ds(off, 16)] = lax.shift_right_logical(u, 2)
        iblk_v[ch, pl.ds(off, 16)] = lax.shift_right_logical(i, 2)
        ul_v[pl.ds(s, 16)] = lax.shift_left(lax.bitwise_and(u, 3), 5)
        il_v[pl.ds(s, 16)] = lax.shift_left(lax.bitwise_and(i, 3), 5)

    bias_cps = []
    for j in range(NCHUNK):
        dst = pl.ds(j * CHUNK, CHUNK)
        bias_cps.append(pltpu.async_copy(ub_hbm.at[uid_v.at[j]],
                                         ubv_v.at[dst], bsem))
        bias_cps.append(pltpu.async_copy(ib_hbm.at[iid_v.at[j]],
                                         ibv_v.at[dst], bsem))

    gb = gb_v[pl.ds(0, 16)]

    def chunk_body(ch, carry):
        cbase = pl.multiple_of(ch * CHUNK, CHUNK)
        cps = [
            pltpu.async_copy(utab_hbm.at[ublk_v.at[ch]], urows_v, sem),
            pltpu.async_copy(itab_hbm.at[iblk_v.at[ch]], irows_v, sem),
        ]
        for cp in cps:
            cp.wait()
        for g in range(GPC):
            i0 = pl.multiple_of(cbase + g * 16, 16)
            li16 = g * 16 + iota16
            lu = ul_v[pl.ds(i0, 16)]
            li = il_v[pl.ds(i0, 16)]
            acc = jnp.zeros((16,), jnp.float32)
            for c in range(EMBED_DIM):
                u = plsc.load_gather(urows_v, [li16, lu + c])
                v = plsc.load_gather(irows_v, [li16, li + c])
                acc = acc + u * v
            out_v[pl.ds(i0, 16)] = acc
        return carry

    lax.fori_loop(0, NCHUNK, chunk_body, 0)

    for cp in bias_cps:
        cp.wait()

    for k in range(BPW // 16):
        s = k * 16
        out_v[pl.ds(s, 16)] = (out_v[pl.ds(s, 16)] + ubv_v[pl.ds(s, 16)]
                               + ibv_v[pl.ds(s, 16)] + gb)

    pltpu.sync_copy(out_v, out_hbm.at[pl.ds(base, BPW)])


def kernel(user_ids, item_ids, user_embedding, item_embedding, user_bias,
           item_bias, global_bias):
    uids = user_ids.astype(jnp.int32).reshape(NW, NCHUNK, CHUNK)
    iids = item_ids.astype(jnp.int32).reshape(NW, NCHUNK, CHUNK)
    utab = user_embedding.reshape(NUM_USERS // 4, 128)
    itab = item_embedding.reshape(NUM_ITEMS // 4, 128)
    ub = user_bias.reshape(-1)
    ib = item_bias.reshape(-1)
    gb = jnp.broadcast_to(global_bias.reshape(-1)[:1], (16,))
    return _mf_sc(uids, iids, utab, itab, ub, ib, gb)


# speedup vs baseline: 1.0796x; 1.0796x over previous
import functools

import jax
import jax.numpy as jnp
from jax import lax
from jax.experimental import pallas as pl
from jax.experimental.pallas import tpu as pltpu
from jax.experimental.pallas import tpu_sc as plsc

NUM_USERS = 1000000
NUM_ITEMS = 100000
EMBED_DIM = 32
BATCH = 16384

NC = 2
NS = 16
NW = NC * NS
BPW = BATCH // NW
C3 = 32
NCH = BPW // C3
GPC = C3 // 16

_mesh = plsc.VectorSubcoreMesh(core_axis_name="c", subcore_axis_name="s")


@functools.partial(
    pl.kernel,
    out_type=jax.ShapeDtypeStruct((BATCH,), jnp.float32),
    mesh=_mesh,
    compiler_params=pltpu.CompilerParams(needs_layout_passes=False),
    scratch_types=[
        pltpu.VMEM((BPW,), jnp.int32),
        pltpu.VMEM((BPW,), jnp.int32),
        pltpu.VMEM((BPW,), jnp.int32),
        pltpu.VMEM((BPW,), jnp.int32),
        pltpu.VMEM((C3, 8, EMBED_DIM), jnp.float32),
        pltpu.VMEM((C3, 8, EMBED_DIM), jnp.float32),
        pltpu.VMEM((C3, 8), jnp.float32),
        pltpu.VMEM((C3, 8), jnp.float32),
        pltpu.VMEM((16,), jnp.float32),
        pltpu.VMEM((BPW,), jnp.float32),
        pltpu.SemaphoreType.DMA,
    ],
)
def _mf_sc(uids_hbm, iids_hbm, utab_hbm, itab_hbm, ub_hbm, ib_hbm, gb_hbm,
           out_hbm, ublk_v, iblk_v, ur_v, ir_v, urows_v, irows_v, ubr_v,
           ibr_v, gb_v, out_v, sem):
    wid = lax.axis_index("s") * NC + lax.axis_index("c")
    base = pl.multiple_of(wid * BPW, BPW)

    utab3 = utab_hbm.reshape(NUM_USERS // 8, 8, EMBED_DIM)
    itab3 = itab_hbm.reshape(NUM_ITEMS // 8, 8, EMBED_DIM)
    ub2 = ub_hbm
    ib2 = ib_hbm

    pltpu.sync_copy(uids_hbm.at[wid], ublk_v)
    pltpu.sync_copy(iids_hbm.at[wid], iblk_v)
    pltpu.sync_copy(gb_hbm, gb_v)

    iota16 = lax.iota(jnp.int32, 16)

    for k in range(BPW // 16):
        s = k * 16
        u = ublk_v[pl.ds(s, 16)]
        i = iblk_v[pl.ds(s, 16)]
        ur_v[pl.ds(s, 16)] = lax.bitwise_and(u, 7)
        ir_v[pl.ds(s, 16)] = lax.bitwise_and(i, 7)
        ublk_v[pl.ds(s, 16)] = lax.shift_right_logical(u, 3)
        iblk_v[pl.ds(s, 16)] = lax.shift_right_logical(i, 3)

    gb = gb_v[pl.ds(0, 16)]

    def chunk_body(ch, carry):
        cbase = pl.multiple_of(ch * C3, C3)
        cps = []
        for g in range(GPC):
            ub16 = ublk_v[pl.ds(cbase + g * 16, 16)]
            ib16 = iblk_v[pl.ds(cbase + g * 16, 16)]
            for l in range(16):
                j = g * 16 + l
                cps.append(pltpu.async_copy(
                    utab3.at[ub16[l]], urows_v.at[j], sem))
                cps.append(pltpu.async_copy(
                    itab3.at[ib16[l]], irows_v.at[j], sem))
                cps.append(pltpu.async_copy(
                    ub2.at[ub16[l]], ubr_v.at[j], sem))
                cps.append(pltpu.async_copy(
                    ib2.at[ib16[l]], ibr_v.at[j], sem))
        for cp in cps:
            cp.wait()
        for g in range(GPC):
            i0 = pl.multiple_of(cbase + g * 16, 16)
            li16 = g * 16 + iota16
            ru = ur_v[pl.ds(i0, 16)]
            ri = ir_v[pl.ds(i0, 16)]
            acc = (plsc.load_gather(ubr_v, [li16, ru])
                   + plsc.load_gather(ibr_v, [li16, ri]) + gb)
            for c in range(EMBED_DIM):
                cc = jnp.full((16,), c, jnp.int32)
                u = plsc.load_gather(urows_v, [li16, ru, cc])
                v = plsc.load_gather(irows_v, [li16, ri, cc])
                acc = acc + u * v
            out_v[pl.ds(i0, 16)] = acc
        return carry

    lax.fori_loop(0, NCH, chunk_body, 0)

    pltpu.sync_copy(out_v, out_hbm.at[pl.ds(base, BPW)])


def kernel(user_ids, item_ids, user_embedding, item_embedding, user_bias,
           item_bias, global_bias):
    uids = user_ids.astype(jnp.int32).reshape(NW, BPW)
    iids = item_ids.astype(jnp.int32).reshape(NW, BPW)
    ub = user_bias.reshape(NUM_USERS // 8, 8)
    ib = item_bias.reshape(NUM_ITEMS // 8, 8)
    gb = jnp.broadcast_to(global_bias.reshape(-1)[:1], (16,))
    return _mf_sc(uids, iids, user_embedding, item_embedding, ub, ib, gb)


# speedup vs baseline: 1.1658x; 1.0799x over previous
import functools

import jax
import jax.numpy as jnp
from jax import lax
from jax.experimental import pallas as pl
from jax.experimental.pallas import tpu as pltpu
from jax.experimental.pallas import tpu_sc as plsc

NUM_USERS = 1000000
NUM_ITEMS = 100000
EMBED_DIM = 32
BATCH = 16384

NC = 2
NS = 16
NW = NC * NS
BPW = BATCH // NW
C3 = 32
NCH = BPW // C3
GPC = C3 // 16

_mesh = plsc.VectorSubcoreMesh(core_axis_name="c", subcore_axis_name="s")


@functools.partial(
    pl.kernel,
    out_type=jax.ShapeDtypeStruct((BATCH,), jnp.float32),
    mesh=_mesh,
    compiler_params=pltpu.CompilerParams(needs_layout_passes=False),
    scratch_types=[
        pltpu.VMEM((BPW,), jnp.int32),
        pltpu.VMEM((BPW,), jnp.int32),
        pltpu.VMEM((BPW,), jnp.int32),
        pltpu.VMEM((BPW,), jnp.int32),
        pltpu.VMEM((C3, EMBED_DIM), jnp.float32),
        pltpu.VMEM((C3, EMBED_DIM), jnp.float32),
        pltpu.VMEM((C3, 8), jnp.float32),
        pltpu.VMEM((C3, 8), jnp.float32),
        pltpu.VMEM((16,), jnp.float32),
        pltpu.VMEM((BPW,), jnp.float32),
        pltpu.SemaphoreType.DMA,
    ],
)
def _mf_sc(uids_hbm, iids_hbm, utab_hbm, itab_hbm, ub_hbm, ib_hbm, gb_hbm,
           out_hbm, ublk_v, iblk_v, ur_v, ir_v, urows_v, irows_v, ubr_v,
           ibr_v, gb_v, out_v, sem):
    wid = lax.axis_index("s") * NC + lax.axis_index("c")
    base = pl.multiple_of(wid * BPW, BPW)

    utab3 = utab_hbm.reshape(NUM_USERS // 8, 8, EMBED_DIM)
    itab3 = itab_hbm.reshape(NUM_ITEMS // 8, 8, EMBED_DIM)
    ub2 = ub_hbm
    ib2 = ib_hbm

    pltpu.sync_copy(uids_hbm.at[wid], ublk_v)
    pltpu.sync_copy(iids_hbm.at[wid], iblk_v)
    pltpu.sync_copy(gb_hbm, gb_v)

    iota16 = lax.iota(jnp.int32, 16)

    for k in range(BPW // 16):
        s = k * 16
        u = ublk_v[pl.ds(s, 16)]
        i = iblk_v[pl.ds(s, 16)]
        ur_v[pl.ds(s, 16)] = lax.bitwise_and(u, 7)
        ir_v[pl.ds(s, 16)] = lax.bitwise_and(i, 7)
        ublk_v[pl.ds(s, 16)] = lax.shift_right_logical(u, 3)
        iblk_v[pl.ds(s, 16)] = lax.shift_right_logical(i, 3)

    gb = gb_v[pl.ds(0, 16)]

    def chunk_body(ch, carry):
        cbase = pl.multiple_of(ch * C3, C3)
        cps = []
        for g in range(GPC):
            ub16 = ublk_v[pl.ds(cbase + g * 16, 16)]
            ib16 = iblk_v[pl.ds(cbase + g * 16, 16)]
            ru16 = ur_v[pl.ds(cbase + g * 16, 16)]
            ri16 = ir_v[pl.ds(cbase + g * 16, 16)]
            for l in range(16):
                j = g * 16 + l
                cps.append(pltpu.async_copy(
                    utab3.at[ub16[l], ru16[l]], urows_v.at[j], sem))
                cps.append(pltpu.async_copy(
                    itab3.at[ib16[l], ri16[l]], irows_v.at[j], sem))
                cps.append(pltpu.async_copy(
                    ub2.at[ub16[l]], ubr_v.at[j], sem))
                cps.append(pltpu.async_copy(
                    ib2.at[ib16[l]], ibr_v.at[j], sem))
        for cp in cps:
            cp.wait()
        for g in range(GPC):
            i0 = pl.multiple_of(cbase + g * 16, 16)
            li16 = g * 16 + iota16
            ru = ur_v[pl.ds(i0, 16)]
            ri = ir_v[pl.ds(i0, 16)]
            acc = (plsc.load_gather(ubr_v, [li16, ru])
                   + plsc.load_gather(ibr_v, [li16, ri]) + gb)
            for c in range(EMBED_DIM):
                cc = jnp.full((16,), c, jnp.int32)
                u = plsc.load_gather(urows_v, [li16, cc])
                v = plsc.load_gather(irows_v, [li16, cc])
                acc = acc + u * v
            out_v[pl.ds(i0, 16)] = acc
        return carry

    lax.fori_loop(0, NCH, chunk_body, 0)

    pltpu.sync_copy(out_v, out_hbm.at[pl.ds(base, BPW)])


def kernel(user_ids, item_ids, user_embedding, item_embedding, user_bias,
           item_bias, global_bias):
    uids = user_ids.astype(jnp.int32).reshape(NW, BPW)
    iids = item_ids.astype(jnp.int32).reshape(NW, BPW)
    ub = user_bias.reshape(NUM_USERS // 8, 8)
    ib = item_bias.reshape(NUM_ITEMS // 8, 8)
    gb = jnp.broadcast_to(global_bias.reshape(-1)[:1], (16,))
    return _mf_sc(uids, iids, user_embedding, item_embedding, ub, ib, gb)
